# baseline (device time: 43827 ns/iter reference)
import jax
import jax.numpy as jnp
from jax import lax
from jax.experimental import pallas as pl
from jax.experimental.pallas import tpu as pltpu

N_DEV = 4
M = 1024
N = 1024
H = M // 2
Q = M // 4
NC = 8
SC = N // NC
GB = N // 4
ORDER = (0, 4, 1, 5, 2, 6, 3, 7)
GORDER = (0, 2, 1, 3)


def kernel(x, w_mat):
    def body(x_ref, w_ref, out_ref, c1_ref, c2_ref, send_sems, recv_sems):
        my = lax.axis_index("i")
        pa = my ^ 1
        pb = 3 - my
        bit_a = (my ^ (my >> 1)) & 1
        bit_b = (my >> 1) & 1
        bit1a = my & 1

        chains = []
        for idx in range(NC):
            if idx < NC // 2:
                chains.append(dict(
                    col=idx * SC, p_half=pa, p_quar=pb,
                    kb=bit_a, qk=2 * bit_a + bit_b, qs=2 * bit_a + (1 - bit_b),
                ))
            else:
                chains.append(dict(
                    col=idx * SC, p_half=pb, p_quar=pa,
                    kb=bit_b, qk=2 * bit_b + bit1a, qs=2 * bit_b + (1 - bit1a),
                ))

        barrier_sem = pltpu.get_barrier_semaphore()
        for nbr in (pa, pb):
            pl.semaphore_signal(
                barrier_sem, inc=1,
                device_id=(nbr,), device_id_type=pl.DeviceIdType.MESH,
            )

        def gemm_block(rh, col):
            out_ref[pl.ds(rh * H, H), pl.ds(col, GB)] = jnp.dot(
                x_ref[pl.ds(rh * H, H), :],
                w_ref[:, pl.ds(col, GB)],
                preferred_element_type=jnp.float32,
            )

        def copy(src, dst, phase, idx, peer):
            return pltpu.make_async_remote_copy(
                src_ref=src, dst_ref=dst,
                send_sem=send_sems.at[phase, idx],
                recv_sem=recv_sems.at[phase, idx],
                device_id=(peer,),
                device_id_type=pl.DeviceIdType.MESH,
            )

        p1 = {}
        first = True
        for g in GORDER:
            ch0 = chains[2 * g]
            gemm_block(1 - ch0["kb"], g * GB)
            if first:
                pl.semaphore_wait(barrier_sem, 2)
                first = False
            for idx in (2 * g, 2 * g + 1):
                ch = chains[idx]
                p1[idx] = copy(
                    out_ref.at[pl.ds((1 - ch["kb"]) * H, H), pl.ds(ch["col"], SC)],
                    c1_ref.at[idx], 0, idx, ch["p_half"])
                p1[idx].start()

        for g in GORDER:
            gemm_block(chains[2 * g]["kb"], g * GB)

        p2 = {}
        for idx in ORDER:
            ch = chains[idx]
            p1[idx].wait_recv()
            out_ref[pl.ds(ch["kb"] * H, H), pl.ds(ch["col"], SC)] += c1_ref[idx]
            p2[idx] = copy(
                out_ref.at[pl.ds(ch["qs"] * Q, Q), pl.ds(ch["col"], SC)],
                c2_ref.at[idx], 1, idx, ch["p_quar"])
            p2[idx].start()

        p3 = {}
        for idx in ORDER:
            ch = chains[idx]
            p2[idx].wait_recv()
            out_ref[pl.ds(ch["qk"] * Q, Q), pl.ds(ch["col"], SC)] += c2_ref[idx]
            p3[idx] = copy(
                out_ref.at[pl.ds(ch["qk"] * Q, Q), pl.ds(ch["col"], SC)],
                out_ref.at[pl.ds(ch["qk"] * Q, Q), pl.ds(ch["col"], SC)],
                2, idx, ch["p_quar"])
            p3[idx].start()

        p4 = {}
        for idx in ORDER:
            ch = chains[idx]
            p3[idx].wait_recv()
            p4[idx] = copy(
                out_ref.at[pl.ds(ch["kb"] * H, H), pl.ds(ch["col"], SC)],
                out_ref.at[pl.ds(ch["kb"] * H, H), pl.ds(ch["col"], SC)],
                3, idx, ch["p_half"])
            p4[idx].start()

        for idx in ORDER:
            p4[idx].wait_recv()
        for rdmas in (p1, p2, p3, p4):
            for idx in ORDER:
                rdmas[idx].wait_send()

    return pl.pallas_call(
        body,
        out_shape=jax.ShapeDtypeStruct((M, N), jnp.float32),
        in_specs=[
            pl.BlockSpec(memory_space=pltpu.VMEM),
            pl.BlockSpec(memory_space=pltpu.VMEM),
        ],
        out_specs=pl.BlockSpec(memory_space=pltpu.VMEM),
        scratch_shapes=[
            pltpu.VMEM((NC, H, SC), jnp.float32),
            pltpu.VMEM((NC, Q, SC), jnp.float32),
            pltpu.SemaphoreType.DMA((4, NC)),
            pltpu.SemaphoreType.DMA((4, NC)),
        ],
        compiler_params=pltpu.CompilerParams(collective_id=0),
    )(x, w_mat)
